# Initial kernel scaffold; baseline (speedup 1.0000x reference)
import jax
import jax.numpy as jnp
from jax.experimental import pallas as pl


def kernel(x):
    raise NotImplementedError("write your pallas kernel here")


# speedup vs baseline: 3.8805x; 3.8805x over previous
import functools

import jax
import jax.numpy as jnp
from jax import lax
from jax.experimental import pallas as pl
from jax.experimental.pallas import tpu as pltpu
from jax.experimental.pallas import tpu_sc as plsc

NROWS = 64
NCOLS = 8192
KACT = 128
L = 16
SLICES = NCOLS // L

_MESH = plsc.VectorSubcoreMesh(core_axis_name="c", subcore_axis_name="s")


def _mkkey(v):
    b = plsc.bitcast(v, jnp.uint32)
    sign = b >> jnp.uint32(31)
    return jnp.where(sign == jnp.uint32(1),
                     b ^ jnp.uint32(0xFFFFFFFF),
                     b | jnp.uint32(0x80000000))


def _scan_hist(hist_ref, rank):
    lane = lax.iota(jnp.int32, L)
    acc = jnp.int32(0)
    bstar = jnp.int32(0)
    rank_new = rank
    nb = jnp.int32(0)
    found = jnp.bool_(False)
    for j in range(15, -1, -1):
        h = hist_ref[pl.ds(j * L, L)]
        hrev = lax.rev(h, (0,))
        c = plsc.cumsum(hrev) + acc
        m = c >= rank
        mi = jnp.where(m, 1, 0)
        anyf = jnp.max(mi) > 0
        p = jnp.max(plsc.all_reduce_ffs(m))
        sel = lane == p
        above = jnp.sum(jnp.where(sel, c - hrev, 0))
        cnt_here = jnp.sum(jnp.where(sel, hrev, 0))
        take = jnp.logical_and(jnp.logical_not(found), anyf)
        bstar = jnp.where(take, j * L + (L - 1) - p, bstar)
        rank_new = jnp.where(take, rank - above, rank_new)
        nb = jnp.where(take, cnt_here, nb)
        found = jnp.logical_or(found, anyf)
        acc = acc + jnp.sum(h)
    return bstar, rank_new, nb


def _zero_hist(hist_ref):
    z = jnp.zeros((L,), jnp.int32)
    for j in range(16):
        hist_ref[pl.ds(j * L, L)] = z


@functools.partial(
    pl.kernel,
    out_type=jax.ShapeDtypeStruct((NROWS * NCOLS,), jnp.float32),
    mesh=_MESH,
    scratch_types=[
        pltpu.VMEM((NCOLS,), jnp.float32),
        pltpu.VMEM((NCOLS,), jnp.int32),
        pltpu.VMEM((NCOLS,), jnp.int32),
        pltpu.VMEM((256,), jnp.int32),
    ],
    compiler_params=pltpu.CompilerParams(needs_layout_passes=False),
)
def _sparsify_sc(x_hbm, out_hbm, xrow, canda, candb, hist):
    wid = lax.axis_index("s") * 2 + lax.axis_index("c")
    ones = jnp.ones((L,), jnp.int32)
    lane = lax.iota(jnp.int32, L)

    def do_row(rr, carry):
        row = wid * 2 + rr
        pltpu.sync_copy(x_hbm.at[pl.ds(row * NCOLS, NCOLS)], xrow)

        _zero_hist(hist)

        def h1(i, carry):
            key = _mkkey(xrow[pl.ds(i * L, L)])
            idx = plsc.bitcast(key >> jnp.uint32(24), jnp.int32)
            plsc.addupdate_scatter(hist, [idx], ones)
            return carry

        lax.fori_loop(0, SLICES, h1, 0)
        bstar, rank, _ = _scan_hist(hist, jnp.int32(KACT))
        bstar_u = bstar.astype(jnp.uint32)

        def c1(i, off):
            key = _mkkey(xrow[pl.ds(i * L, L)])
            m = (key >> jnp.uint32(24)) == bstar_u
            mi = jnp.where(m, 1, 0)
            pos = off + plsc.cumsum(mi) - 1
            plsc.store_scatter(canda, [pos], plsc.bitcast(key, jnp.int32),
                               mask=m)
            return off + jnp.sum(mi)

        n = lax.fori_loop(0, SLICES, c1, jnp.int32(0))
        prefix = jnp.uint32(bstar_u << jnp.uint32(24))

        src, dst = canda, candb
        for shift in (16, 8, 0):
            sh = jnp.uint32(shift)
            nsl = (n + (L - 1)) // L
            _zero_hist(hist)

            def hr(i, carry, src=src, sh=sh):
                key = plsc.bitcast(src[pl.ds(i * L, L)], jnp.uint32)
                idx = plsc.bitcast((key >> sh) & jnp.uint32(0xFF), jnp.int32)
                valid = i * L + lane < carry
                plsc.addupdate_scatter(hist, [idx], ones, mask=valid)
                return carry

            lax.fori_loop(0, nsl, hr, n)
            bstar, rank, nb = _scan_hist(hist, rank)
            bstar_u = bstar.astype(jnp.uint32)
            prefix = prefix | jnp.uint32(bstar_u << sh)

            if shift > 0:
                def cr(i, off, src=src, dst=dst, sh=sh, bstar_u=bstar_u,
                       n=n):
                    key = plsc.bitcast(src[pl.ds(i * L, L)], jnp.uint32)
                    m = jnp.logical_and(((key >> sh) & jnp.uint32(0xFF))
                                        == bstar_u,
                                        i * L + lane < n)
                    mi = jnp.where(m, 1, 0)
                    pos = off + plsc.cumsum(mi) - 1
                    plsc.store_scatter(dst, [pos],
                                       plsc.bitcast(key, jnp.int32), mask=m)
                    return off + jnp.sum(mi)

                n = lax.fori_loop(0, nsl, cr, jnp.int32(0))
                src, dst = dst, src

        thr_bits = jnp.where(prefix >= jnp.uint32(0x80000000),
                             prefix ^ jnp.uint32(0x80000000),
                             prefix ^ jnp.uint32(0xFFFFFFFF))
        thr = lax.bitcast_convert_type(thr_bits, jnp.float32)

        def fbody(i, carry):
            v = xrow[pl.ds(i * L, L)]
            xrow[pl.ds(i * L, L)] = jnp.where(v >= thr, v, jnp.float32(0.0))
            return carry

        lax.fori_loop(0, SLICES, fbody, 0)
        pltpu.sync_copy(xrow, out_hbm.at[pl.ds(row * NCOLS, NCOLS)])
        return carry

    lax.fori_loop(0, 2, do_row, 0)


@jax.jit
def kernel(x):
    out = _sparsify_sc(x.reshape(-1))
    return out.reshape(NROWS, NCOLS)


# speedup vs baseline: 4.6621x; 1.2014x over previous
import functools

import jax
import jax.numpy as jnp
from jax import lax
from jax.experimental import pallas as pl
from jax.experimental.pallas import tpu as pltpu
from jax.experimental.pallas import tpu_sc as plsc

NROWS = 64
NCOLS = 8192
KACT = 128
L = 16
SLICES = NCOLS // L

_MESH = plsc.VectorSubcoreMesh(core_axis_name="c", subcore_axis_name="s")


def _mkkey(v):
    b = plsc.bitcast(v, jnp.uint32)
    sign = b >> jnp.uint32(31)
    return jnp.where(sign == jnp.uint32(1),
                     b ^ jnp.uint32(0xFFFFFFFF),
                     b | jnp.uint32(0x80000000))


def _scan_hist(hist_ref, rank):
    lane = lax.iota(jnp.int32, L)
    comb = jnp.full((L,), -1, jnp.int32)
    acc = jnp.int32(0)
    for j in range(15, -1, -1):
        h = hist_ref[pl.ds(j * L, L)]
        hrev = lax.rev(h, (0,))
        c = plsc.cumsum(hrev) + acc
        m = c >= rank
        bid = (j * L + L - 1) - lane
        cand = jnp.where(m, (bid << 16) | (c - hrev), -1)
        comb = jnp.maximum(comb, cand)
        acc = acc + jnp.sum(h)
    best = jnp.max(comb)
    bstar = best >> 16
    above = best & 0xFFFF
    return bstar, rank - above


def _zero_hist(hist_ref):
    z = jnp.zeros((L,), jnp.int32)
    for j in range(16):
        hist_ref[pl.ds(j * L, L)] = z


@functools.partial(
    pl.kernel,
    out_type=jax.ShapeDtypeStruct((NROWS * NCOLS,), jnp.float32),
    mesh=_MESH,
    scratch_types=[
        pltpu.VMEM((NCOLS,), jnp.float32),
        pltpu.VMEM((NCOLS,), jnp.int32),
        pltpu.VMEM((NCOLS,), jnp.int32),
        pltpu.VMEM((256,), jnp.int32),
    ],
    compiler_params=pltpu.CompilerParams(needs_layout_passes=False),
)
def _sparsify_sc(x_hbm, out_hbm, xrow, canda, candb, hist):
    wid = lax.axis_index("s") * 2 + lax.axis_index("c")
    ones = jnp.ones((L,), jnp.int32)
    lane = lax.iota(jnp.int32, L)

    def do_row(rr, carry):
        row = wid * 2 + rr
        pltpu.sync_copy(x_hbm.at[pl.ds(row * NCOLS, NCOLS)], xrow)

        _zero_hist(hist)

        def h1(i, carry):
            key = _mkkey(xrow[pl.ds(i * L, L)])
            idx = plsc.bitcast(key >> jnp.uint32(24), jnp.int32)
            plsc.addupdate_scatter(hist, [idx], ones)
            return carry

        lax.fori_loop(0, SLICES, h1, 0, unroll=8)
        bstar, rank = _scan_hist(hist, jnp.int32(KACT))
        bstar_u = bstar.astype(jnp.uint32)

        def c1(i, offv):
            key = _mkkey(xrow[pl.ds(i * L, L)])
            m = (key >> jnp.uint32(24)) == bstar_u
            mi = jnp.where(m, 1, 0)
            pos = offv + plsc.cumsum(mi) - 1
            plsc.store_scatter(canda, [pos], plsc.bitcast(key, jnp.int32),
                               mask=m)
            return offv + plsc.all_reduce_population_count(m)

        offv = lax.fori_loop(0, SLICES, c1, jnp.zeros((L,), jnp.int32),
                             unroll=4)
        nv = offv
        n = jnp.max(nv)
        prefix = jnp.uint32(bstar_u << jnp.uint32(24))

        src, dst = canda, candb
        for shift in (16, 8, 0):
            sh = jnp.uint32(shift)
            nsl = (n + (L - 1)) // L
            _zero_hist(hist)

            def hr(i, carry, src=src, sh=sh, nv=nv):
                key = plsc.bitcast(src[pl.ds(i * L, L)], jnp.uint32)
                idx = plsc.bitcast((key >> sh) & jnp.uint32(0xFF), jnp.int32)
                valid = i * L + lane < nv
                plsc.addupdate_scatter(hist, [idx], ones, mask=valid)
                return carry

            lax.fori_loop(0, nsl, hr, 0)
            bstar, rank = _scan_hist(hist, rank)
            bstar_u = bstar.astype(jnp.uint32)
            prefix = prefix | jnp.uint32(bstar_u << sh)

            if shift > 0:
                def cr(i, offv, src=src, dst=dst, sh=sh, bstar_u=bstar_u,
                       nv=nv):
                    key = plsc.bitcast(src[pl.ds(i * L, L)], jnp.uint32)
                    m = jnp.logical_and(((key >> sh) & jnp.uint32(0xFF))
                                        == bstar_u,
                                        i * L + lane < nv)
                    mi = jnp.where(m, 1, 0)
                    pos = offv + plsc.cumsum(mi) - 1
                    plsc.store_scatter(dst, [pos],
                                       plsc.bitcast(key, jnp.int32), mask=m)
                    return offv + plsc.all_reduce_population_count(m)

                offv = lax.fori_loop(0, nsl, cr, jnp.zeros((L,), jnp.int32))
                nv = offv
                n = jnp.max(nv)
                src, dst = dst, src

        thr_bits = jnp.where(prefix >= jnp.uint32(0x80000000),
                             prefix ^ jnp.uint32(0x80000000),
                             prefix ^ jnp.uint32(0xFFFFFFFF))
        thr = lax.bitcast_convert_type(thr_bits, jnp.float32)

        def fbody(i, carry):
            v = xrow[pl.ds(i * L, L)]
            xrow[pl.ds(i * L, L)] = jnp.where(v >= thr, v, jnp.float32(0.0))
            return carry

        lax.fori_loop(0, SLICES, fbody, 0, unroll=8)
        pltpu.sync_copy(xrow, out_hbm.at[pl.ds(row * NCOLS, NCOLS)])
        return carry

    lax.fori_loop(0, 2, do_row, 0)


@jax.jit
def kernel(x):
    out = _sparsify_sc(x.reshape(-1))
    return out.reshape(NROWS, NCOLS)


# speedup vs baseline: 5.7822x; 1.2403x over previous
import functools

import jax
import jax.numpy as jnp
from jax import lax
from jax.experimental import pallas as pl
from jax.experimental.pallas import tpu as pltpu
from jax.experimental.pallas import tpu_sc as plsc

NROWS = 64
NCOLS = 8192
KACT = 128
L = 16
SLICES = NCOLS // L

_MESH = plsc.VectorSubcoreMesh(core_axis_name="c", subcore_axis_name="s")


def _mkkey(v):
    b = plsc.bitcast(v, jnp.uint32)
    sign = b >> jnp.uint32(31)
    return jnp.where(sign == jnp.uint32(1),
                     b ^ jnp.uint32(0xFFFFFFFF),
                     b | jnp.uint32(0x80000000))


def _scan_hist(hist_ref, hbase, rank):
    lane = lax.iota(jnp.int32, L)
    comb = jnp.full((L,), -1, jnp.int32)
    acc = jnp.int32(0)
    for j in range(15, -1, -1):
        h = hist_ref[pl.ds(hbase + j * L, L)]
        hrev = lax.rev(h, (0,))
        c = plsc.cumsum(hrev) + acc
        m = c >= rank
        bid = (j * L + L - 1) - lane
        cand = jnp.where(m, (bid << 16) | (c - hrev), -1)
        comb = jnp.maximum(comb, cand)
        acc = acc + jnp.sum(h)
    best = jnp.max(comb)
    bstar = best >> 16
    above = best & 0xFFFF
    return bstar, rank - above


def _zero_hist(hist_ref):
    z = jnp.zeros((L,), jnp.int32)
    for j in range(32):
        hist_ref[pl.ds(j * L, L)] = z


@functools.partial(
    pl.kernel,
    out_type=jax.ShapeDtypeStruct((NROWS * NCOLS,), jnp.float32),
    mesh=_MESH,
    scratch_types=[
        pltpu.VMEM((2 * NCOLS,), jnp.float32),
        pltpu.VMEM((2 * NCOLS,), jnp.int32),
        pltpu.VMEM((2 * NCOLS,), jnp.int32),
        pltpu.VMEM((512,), jnp.int32),
    ],
    compiler_params=pltpu.CompilerParams(needs_layout_passes=False),
)
def _sparsify_sc(x_hbm, out_hbm, xrow, canda, candb, hist):
    wid = lax.axis_index("s") * 2 + lax.axis_index("c")
    ones = jnp.ones((L,), jnp.int32)
    lane = lax.iota(jnp.int32, L)
    u24 = jnp.uint32(24)

    base_hbm = wid * (2 * NCOLS)
    pltpu.sync_copy(x_hbm.at[pl.ds(base_hbm, 2 * NCOLS)], xrow)

    _zero_hist(hist)

    def h1(i, carry):
        k0 = _mkkey(xrow[pl.ds(i * L, L)])
        k1 = _mkkey(xrow[pl.ds(NCOLS + i * L, L)])
        i0 = plsc.bitcast(k0 >> u24, jnp.int32)
        i1 = plsc.bitcast(k1 >> u24, jnp.int32) + 256
        plsc.addupdate_scatter(hist, [i0], ones)
        plsc.addupdate_scatter(hist, [i1], ones)
        return carry

    lax.fori_loop(0, SLICES, h1, 0, unroll=8)
    bstar0, rank0 = _scan_hist(hist, 0, jnp.int32(KACT))
    bstar1, rank1 = _scan_hist(hist, 256, jnp.int32(KACT))
    bu0 = bstar0.astype(jnp.uint32)
    bu1 = bstar1.astype(jnp.uint32)

    def c1(i, offs):
        off0, off1 = offs
        k0 = _mkkey(xrow[pl.ds(i * L, L)])
        k1 = _mkkey(xrow[pl.ds(NCOLS + i * L, L)])
        m0 = (k0 >> u24) == bu0
        m1 = (k1 >> u24) == bu1
        p0 = off0 + plsc.cumsum(jnp.where(m0, 1, 0)) - 1
        p1 = off1 + plsc.cumsum(jnp.where(m1, 1, 0)) - 1
        plsc.store_scatter(canda, [p0], plsc.bitcast(k0, jnp.int32), mask=m0)
        plsc.store_scatter(canda, [p1 + NCOLS],
                           plsc.bitcast(k1, jnp.int32), mask=m1)
        return (off0 + plsc.all_reduce_population_count(m0),
                off1 + plsc.all_reduce_population_count(m1))

    z16 = jnp.zeros((L,), jnp.int32)
    nv0, nv1 = lax.fori_loop(0, SLICES, c1, (z16, z16), unroll=4)
    prefix0 = jnp.uint32(bu0 << u24)
    prefix1 = jnp.uint32(bu1 << u24)

    src, dst = canda, candb
    for shift in (16, 8, 0):
        sh = jnp.uint32(shift)
        nsl = (jnp.maximum(jnp.max(nv0), jnp.max(nv1)) + (L - 1)) // L
        _zero_hist(hist)

        def hr(i, carry, src=src, sh=sh, nv0=nv0, nv1=nv1):
            k0 = plsc.bitcast(src[pl.ds(i * L, L)], jnp.uint32)
            k1 = plsc.bitcast(src[pl.ds(NCOLS + i * L, L)], jnp.uint32)
            i0 = plsc.bitcast((k0 >> sh) & jnp.uint32(0xFF), jnp.int32)
            i1 = plsc.bitcast((k1 >> sh) & jnp.uint32(0xFF), jnp.int32) + 256
            iv = i * L + lane
            plsc.addupdate_scatter(hist, [i0], ones, mask=iv < nv0)
            plsc.addupdate_scatter(hist, [i1], ones, mask=iv < nv1)
            return carry

        lax.fori_loop(0, nsl, hr, 0)
        bstar0, rank0 = _scan_hist(hist, 0, rank0)
        bstar1, rank1 = _scan_hist(hist, 256, rank1)
        bu0 = bstar0.astype(jnp.uint32)
        bu1 = bstar1.astype(jnp.uint32)
        prefix0 = prefix0 | jnp.uint32(bu0 << sh)
        prefix1 = prefix1 | jnp.uint32(bu1 << sh)

        if shift > 0:
            def cr(i, offs, src=src, dst=dst, sh=sh, bu0=bu0, bu1=bu1,
                   nv0=nv0, nv1=nv1):
                off0, off1 = offs
                k0 = plsc.bitcast(src[pl.ds(i * L, L)], jnp.uint32)
                k1 = plsc.bitcast(src[pl.ds(NCOLS + i * L, L)], jnp.uint32)
                iv = i * L + lane
                m0 = jnp.logical_and(((k0 >> sh) & jnp.uint32(0xFF)) == bu0,
                                     iv < nv0)
                m1 = jnp.logical_and(((k1 >> sh) & jnp.uint32(0xFF)) == bu1,
                                     iv < nv1)
                p0 = off0 + plsc.cumsum(jnp.where(m0, 1, 0)) - 1
                p1 = off1 + plsc.cumsum(jnp.where(m1, 1, 0)) - 1
                plsc.store_scatter(dst, [p0], plsc.bitcast(k0, jnp.int32),
                                   mask=m0)
                plsc.store_scatter(dst, [p1 + NCOLS],
                                   plsc.bitcast(k1, jnp.int32), mask=m1)
                return (off0 + plsc.all_reduce_population_count(m0),
                        off1 + plsc.all_reduce_population_count(m1))

            nv0, nv1 = lax.fori_loop(0, nsl, cr, (z16, z16))
            src, dst = dst, src

    def unkey(prefix):
        bits = jnp.where(prefix >= jnp.uint32(0x80000000),
                         prefix ^ jnp.uint32(0x80000000),
                         prefix ^ jnp.uint32(0xFFFFFFFF))
        return lax.bitcast_convert_type(bits, jnp.float32)

    thr0 = unkey(prefix0)
    thr1 = unkey(prefix1)

    def fbody(i, carry):
        v0 = xrow[pl.ds(i * L, L)]
        v1 = xrow[pl.ds(NCOLS + i * L, L)]
        xrow[pl.ds(i * L, L)] = jnp.where(v0 >= thr0, v0, jnp.float32(0.0))
        xrow[pl.ds(NCOLS + i * L, L)] = jnp.where(v1 >= thr1, v1,
                                                  jnp.float32(0.0))
        return carry

    lax.fori_loop(0, SLICES, fbody, 0, unroll=8)
    pltpu.sync_copy(xrow, out_hbm.at[pl.ds(base_hbm, 2 * NCOLS)])


@jax.jit
def kernel(x):
    out = _sparsify_sc(x.reshape(-1))
    return out.reshape(NROWS, NCOLS)
